# baseline (speedup 1.0000x reference)
import functools

import jax
import jax.numpy as jnp
from jax import lax
from jax.experimental import pallas as pl
from jax.experimental.pallas import tpu as pltpu
from jax.experimental.pallas import tpu_sc as plsc


def kernel(node_indices, embeddings):
  B, = node_indices.shape
  V, D = embeddings.shape

  info = plsc.get_sparse_core_info()
  NC, NS = info.num_cores, info.num_subcores
  NW = NC * NS
  assert B % NW == 0
  b_per_w = B // NW
  chunk = min(128, b_per_w)
  n_chunks = b_per_w // chunk

  mesh = plsc.VectorSubcoreMesh(core_axis_name="c", subcore_axis_name="s")

  @functools.partial(
      pl.kernel,
      out_type=jax.ShapeDtypeStruct((B, D), jnp.float32),
      mesh=mesh,
      compiler_params=pltpu.CompilerParams(use_tc_tiling_on_sc=False),
      scratch_types=[
          pltpu.VMEM((b_per_w,), jnp.int32),
          pltpu.VMEM((b_per_w, D), jnp.float32),
          pltpu.SemaphoreType.DMA,
      ],
  )
  def gather_kernel(idx_hbm, table_hbm, out_hbm, idx_v, rows_v, sem):
    wid = lax.axis_index("s") * NC + lax.axis_index("c")
    base = wid * b_per_w
    pltpu.sync_copy(idx_hbm.at[pl.ds(base, b_per_w)], idx_v)
    copies = []
    for j in range(n_chunks):
      copies.append(
          pltpu.async_copy(
              table_hbm.at[idx_v.at[pl.ds(j * chunk, chunk)]],
              rows_v.at[pl.ds(j * chunk, chunk)],
              sem,
          )
      )
    for c in copies:
      c.wait()
    pltpu.sync_copy(rows_v, out_hbm.at[pl.ds(base, b_per_w)])

  return gather_kernel(node_indices.astype(jnp.int32), embeddings)


# speedup vs baseline: 4.7428x; 4.7428x over previous
import functools

import jax
import jax.numpy as jnp
from jax import lax
from jax.experimental import pallas as pl
from jax.experimental.pallas import tpu as pltpu
from jax.experimental.pallas import tpu_sc as plsc


def kernel(node_indices, embeddings):
  B, = node_indices.shape
  V, D = embeddings.shape
  L = 16
  LANES = 128

  info = plsc.get_sparse_core_info()
  NC, NS = info.num_cores, info.num_subcores
  NW = NC * NS
  assert B % NW == 0
  b_per_w = B // NW
  NBUF = 8
  n_chunks = b_per_w // NBUF
  assert b_per_w % NBUF == 0

  mesh = plsc.VectorSubcoreMesh(core_axis_name="c", subcore_axis_name="s")

  @functools.partial(
      pl.kernel,
      out_type=jax.ShapeDtypeStruct((D, B), jnp.float32),
      mesh=mesh,
      compiler_params=pltpu.CompilerParams(
          use_tc_tiling_on_sc=True, needs_layout_passes=False
      ),
      scratch_types=[
          pltpu.VMEM((b_per_w,), jnp.int32),
          pltpu.VMEM((2, NBUF, D, LANES), jnp.float32),
          pltpu.VMEM((D, b_per_w), jnp.float32),
          pltpu.SemaphoreType.DMA,
      ],
  )
  def gather_kernel(idx_hbm, table_hbm, out_hbm, idx_v, stage_v, cols_v, sem):
    wid = lax.axis_index("s") * NC + lax.axis_index("c")
    base = wid * b_per_w
    pltpu.sync_copy(idx_hbm.at[pl.ds(base, b_per_w)], idx_v)

    jlo = lax.iota(jnp.int32, L)
    jhi = jlo + L

    def chunk_idx(k):
      idx16 = idx_v[pl.ds((k >> 1) * (2 * NBUF), 2 * NBUF)]
      odd = k & 1
      return [
          jnp.where(odd != 0, idx16[NBUF + s], idx16[s]) for s in range(NBUF)
      ]

    def fire(k, half):
      cs = chunk_idx(k)
      for s in range(NBUF):
        c = cs[s]
        g = pl.multiple_of((c >> 7) << 7, LANES)
        pltpu.async_copy(
            table_hbm.at[:, pl.ds(g, LANES)], stage_v.at[half, s], sem
        )

    def drain_and_extract(k, half):
      col0 = k * NBUF
      cs = chunk_idx(k)
      for s in range(NBUF):
        pltpu.make_async_copy(
            table_hbm.at[:, pl.ds(0, LANES)], stage_v.at[half, s], sem
        ).wait()
        lane = cs[s] & (LANES - 1)
        lane_b = jnp.full((L,), lane, jnp.int32)
        half_b = jnp.full((L,), half, jnp.int32)
        slot_b = jnp.full((L,), s, jnp.int32)
        col_b = jnp.full((L,), col0 + s, jnp.int32)
        v0 = plsc.load_gather(stage_v, [half_b, slot_b, jlo, lane_b])
        v1 = plsc.load_gather(stage_v, [half_b, slot_b, jhi, lane_b])
        plsc.store_scatter(cols_v, [jlo, col_b], v0)
        plsc.store_scatter(cols_v, [jhi, col_b], v1)

    fire(0, 0)

    def chunk_body(k, carry):
      half = lax.rem(k, 2)
      @pl.when(k + 1 < n_chunks)
      def _():
        fire(k + 1, 1 - half)
      drain_and_extract(k, half)
      return carry

    lax.fori_loop(0, n_chunks, chunk_body, 0)

    pltpu.sync_copy(cols_v, out_hbm.at[:, pl.ds(base, b_per_w)])

  out_t = gather_kernel(node_indices.astype(jnp.int32), embeddings.T)
  return out_t.T
